# Initial kernel scaffold; baseline (speedup 1.0000x reference)
import functools
import jax
import jax.numpy as jnp
from jax import lax
from jax.experimental import pallas as pl
from jax.experimental.pallas import tpu as pltpu
from jax.experimental.pallas import tpu_sc as plsc

N = 10000
E = 160000
D = 256
ED = 16
H = 4
C = 256
HC = 1024
B = 8
NM = 3
AD = 64
GH = 256

NPAD = 10240
DUMMY = NPAD - 1
E2 = E + N
NC = 2
NS = 16
T = 128
E2P = 172032
PW1 = E2P // (NC * NS)
NB1 = PW1 // T
PW2 = E2P // NS
NB2 = PW2 // T
ROWS_PER_TILE = NPAD // NS


EP1 = 160256


def _colmean_body(x_ref, o_ref):
    @pl.when(pl.program_id(0) == 0)
    def _():
        o_ref[...] = jnp.zeros_like(o_ref)

    o_ref[0:1, :] += jnp.sum(x_ref[...], axis=0, keepdims=True)

    @pl.when(pl.program_id(0) == EP1 // 512 - 1)
    def _():
        o_ref[...] = o_ref[...] * (1.0 / E)


def _colmean(ea128):
    return pl.pallas_call(
        _colmean_body,
        grid=(EP1 // 512,),
        in_specs=[pl.BlockSpec((512, 128), lambda i: (i, 0))],
        out_specs=pl.BlockSpec((8, 128), lambda i: (0, 0)),
        out_shape=jax.ShapeDtypeStruct((8, 128), jnp.float32),
    )(ea128)


def _proj_body(x_ref, w_ref, o_ref):
    @pl.when(pl.program_id(2) == 0)
    def _():
        o_ref[...] = jnp.zeros_like(o_ref)

    o_ref[0] += jnp.dot(x_ref[0], w_ref[...],
                        preferred_element_type=jnp.float32)


def _proj(xc, w, nch):
    kc, mp, _ = xc.shape
    return pl.pallas_call(
        _proj_body,
        grid=(nch, mp // 256, kc),
        in_specs=[
            pl.BlockSpec((1, 256, 128), lambda ct, nt, kt: (kt, nt, 0)),
            pl.BlockSpec((128, 128), lambda ct, nt, kt: (kt, ct)),
        ],
        out_specs=pl.BlockSpec((1, 256, 128), lambda ct, nt, kt: (ct, nt, 0)),
        out_shape=jax.ShapeDtypeStruct((nch, mp, 128), jnp.float32),
    )(xc, w)


def _elu_body(x_ref, b_ref, o_ref):
    t = x_ref[...] + b_ref[...]
    o_ref[...] = jnp.where(t > 0, t, jnp.exp(t) - 1.0)


def _elu(xc, br):
    return pl.pallas_call(
        _elu_body,
        grid=(8, NPAD // 512),
        in_specs=[
            pl.BlockSpec((1, 512, 128), lambda c, n: (c, n, 0)),
            pl.BlockSpec((1, 1, 128), lambda c, n: (c, 0, 0)),
        ],
        out_specs=pl.BlockSpec((1, 512, 128), lambda c, n: (c, n, 0)),
        out_shape=jax.ShapeDtypeStruct((8, NPAD, 128), jnp.float32),
    )(xc, br)


def _rden_body(a_ref, b_ref, o_ref):
    o_ref[...] = 1.0 / (a_ref[...] + b_ref[...])


def _rden(d0, d1):
    return pl.pallas_call(
        _rden_body,
        grid=(NPAD // 8 // 256,),
        in_specs=[
            pl.BlockSpec((256, 128), lambda i: (i, 0)),
            pl.BlockSpec((256, 128), lambda i: (i, 0)),
        ],
        out_specs=pl.BlockSpec((256, 128), lambda i: (i, 0)),
        out_shape=jax.ShapeDtypeStruct((NPAD // 8, 128), jnp.float32),
    )(d0, d1)


def _pool_body(y_ref, b2_ref, bat_ref, wg1_ref, bg1_ref, wg2_ref, bg2_ref,
               wnp_ref, bnp_ref, gmm_ref, npl_ref, psum, cnt):
    n = pl.program_id(0)

    @pl.when(n == 0)
    def _():
        psum[...] = jnp.zeros_like(psum)
        cnt[...] = jnp.zeros_like(cnt)

    yb = y_ref[...] + b2_ref[...]
    gidx = lax.broadcasted_iota(jnp.int32, (512, 128), 1)
    ohb = (bat_ref[...] == gidx).astype(jnp.float32)
    psum[...] += lax.dot_general(ohb, yb, (((0,), (1,)), ((), ())))
    cnt[...] += lax.dot_general(ohb, jnp.ones((512, 128), jnp.float32),
                                (((0,), (0,)), ((), ())))

    @pl.when(n == NPAD // 512 - 1)
    def _():
        ps = psum[...].reshape(128, HC)
        c1 = cnt[:, 0:1]
        pooled = ps * (1.0 / jnp.maximum(c1, 1.0))
        hh = jnp.maximum(
            jnp.dot(pooled, wg1_ref[...],
                    preferred_element_type=jnp.float32) + bg1_ref[...], 0.0)
        gmm = jnp.dot(hh, wg2_ref[...],
                      preferred_element_type=jnp.float32) + bg2_ref[...]
        npl = jnp.dot(pooled, wnp_ref[...],
                      preferred_element_type=jnp.float32) + bnp_ref[...]
        gmm_ref[...] = gmm[0:8]
        npl_ref[...] = npl[0:8]


def _pool(y2c, b2r, bat2d, wg1, bg1r, wg2p, bg2r, wnpp, bnpr):
    return pl.pallas_call(
        _pool_body,
        grid=(NPAD // 512,),
        in_specs=[
            pl.BlockSpec((8, 512, 128), lambda n: (0, n, 0)),
            pl.BlockSpec((8, 1, 128), lambda n: (0, 0, 0)),
            pl.BlockSpec((512, 128), lambda n: (n, 0)),
            pl.BlockSpec((HC, GH), lambda n: (0, 0)),
            pl.BlockSpec((1, GH), lambda n: (0, 0)),
            pl.BlockSpec((GH, 128), lambda n: (0, 0)),
            pl.BlockSpec((1, 128), lambda n: (0, 0)),
            pl.BlockSpec((HC, 128), lambda n: (0, 0)),
            pl.BlockSpec((1, 128), lambda n: (0, 0)),
        ],
        out_specs=[
            pl.BlockSpec((8, 128), lambda n: (0, 0)),
            pl.BlockSpec((8, 128), lambda n: (0, 0)),
        ],
        out_shape=[
            jax.ShapeDtypeStruct((8, 128), jnp.float32),
            jax.ShapeDtypeStruct((8, 128), jnp.float32),
        ],
        scratch_shapes=[
            pltpu.VMEM((128, 8, 128), jnp.float32),
            pltpu.VMEM((128, 128), jnp.float32),
        ],
    )(y2c, b2r, bat2d, wg1, bg1r, wg2p, bg2r, wnpp, bnpr)


def _pass1_body(xlc, xrc, eec, srch, dsth, atth, ex_out, den_out,
                srcv, dstv, bufa, bufb, bufe, exb, alph, attv, zb,
                sema, semb):
    ci = lax.axis_index("c")
    sid = lax.axis_index("s")
    wid = sid * NC + ci
    iota16 = lax.broadcasted_iota(jnp.int32, (16,), 0)
    zeros16 = jnp.zeros((16,), jnp.float32)

    def zrow(r, _):
        zb[r, :] = zeros16
        return 0

    lax.fori_loop(0, T, zrow, 0)
    for j in range(ROWS_PER_TILE // T):
        pltpu.sync_copy(
            zb, den_out.at[ci].at[pl.ds(sid * ROWS_PER_TILE + j * T, T)])
    pltpu.sync_copy(atth, attv)
    plsc.subcore_barrier()

    def batch(g, _):
        e0 = wid * PW1 + g * T
        pltpu.sync_copy(srch.at[pl.ds(e0, T)], srcv)
        pltpu.sync_copy(dsth.at[pl.ds(e0, T)], dstv)

        def za(e, _):
            alph[e, :] = zeros16
            return 0

        lax.fori_loop(0, T, za, 0)

        for ch in range(8):
            h = ch // 2
            cpa = pltpu.async_copy(xlc.at[ch].at[srcv], bufa, sema)
            cpb = pltpu.async_copy(xrc.at[ch].at[dstv], bufb, semb)
            pltpu.sync_copy(eec.at[ch].at[pl.ds(e0, T)], bufe)
            cpa.wait()
            cpb.wait()
            attc = [attv[h, pl.ds((ch % 2) * 128 + 16 * v, 16)]
                    for v in range(8)]
            hmask = iota16 == h

            def ebody(e, _):
                acc = zeros16
                for v in range(8):
                    sl = pl.ds(16 * v, 16)
                    t = bufa[e, sl] + bufb[e, sl] + bufe[e, sl]
                    m = jnp.maximum(t, 0.2 * t)
                    acc = acc + m * attc[v]
                s = jnp.sum(acc)
                alph[e, :] = alph[e, :] + jnp.where(hmask, s, 0.0)
                return 0

            lax.fori_loop(0, T, ebody, 0)

        def xbody(e, _):
            exb[e, :] = jnp.exp(jnp.minimum(alph[e, :], 60.0))
            return 0

        lax.fori_loop(0, T, xbody, 0)
        pltpu.sync_copy(exb, ex_out.at[pl.ds(e0, T)])
        pltpu.sync_copy(exb, den_out.at[ci].at[dstv], add=True)
        return 0

    lax.fori_loop(0, NB1, batch, 0)


def _pass1(xlc, xrc, eec, src, dst, att):
    f = pl.kernel(
        _pass1_body,
        out_type=[
            jax.ShapeDtypeStruct((E2P, 16), jnp.float32),
            jax.ShapeDtypeStruct((NC, NPAD, 16), jnp.float32),
        ],
        mesh=plsc.VectorSubcoreMesh(
            core_axis_name="c", subcore_axis_name="s",
            num_cores=NC, num_subcores=NS),
        scratch_types=[
            pltpu.VMEM((T,), jnp.int32),
            pltpu.VMEM((T,), jnp.int32),
            pltpu.VMEM((T, 128), jnp.float32),
            pltpu.VMEM((T, 128), jnp.float32),
            pltpu.VMEM((T, 128), jnp.float32),
            pltpu.VMEM((T, 16), jnp.float32),
            pltpu.VMEM((T, 16), jnp.float32),
            pltpu.VMEM((H, C), jnp.float32),
            pltpu.VMEM((T, 16), jnp.float32),
            pltpu.SemaphoreType.DMA,
            pltpu.SemaphoreType.DMA,
        ],
    )
    return f(xlc, xrc, eec, src, dst, att)


def _pass2_body(xlc, srch, dsth, exh, rdenh, out_hbm,
                srcv, dstv, buf, exb, rdb, zb, acc_sh, sem):
    ci = lax.axis_index("c")
    sid = lax.axis_index("s")
    iota16 = lax.broadcasted_iota(jnp.int32, (16,), 0)
    zeros16 = jnp.zeros((16,), jnp.float32)

    def zrow(r, _):
        for v in range(8):
            zb[r, pl.ds(16 * v, 16)] = zeros16
        return 0

    lax.fori_loop(0, T, zrow, 0)

    for cc in range(4):
        ch = ci * 4 + cc
        h = ch // 2
        hmask = iota16 == h
        for j in range(ROWS_PER_TILE // T):
            pltpu.sync_copy(
                zb, acc_sh.at[pl.ds(sid * ROWS_PER_TILE + j * T, T)])
        plsc.subcore_barrier()

        def batch(g, _):
            e0 = sid * PW2 + g * T
            pltpu.sync_copy(srch.at[pl.ds(e0, T)], srcv)
            pltpu.sync_copy(dsth.at[pl.ds(e0, T)], dstv)
            pltpu.sync_copy(exh.at[pl.ds(e0, T)], exb)
            pltpu.async_copy(rdenh.at[dstv], rdb, sem).wait()
            pltpu.async_copy(xlc.at[ch].at[srcv], buf, sem).wait()

            def ebody(e, _):
                p = exb[e, :] * rdb[e, :]
                a = jnp.sum(jnp.where(hmask, p, 0.0))
                for v in range(8):
                    sl = pl.ds(16 * v, 16)
                    buf[e, sl] = buf[e, sl] * a
                return 0

            lax.fori_loop(0, T, ebody, 0)
            pltpu.sync_copy(buf, acc_sh.at[dstv], add=True)
            return 0

        lax.fori_loop(0, NB2, batch, 0)
        plsc.subcore_barrier()
        for j in range(ROWS_PER_TILE // T):
            r0 = sid * ROWS_PER_TILE + j * T
            pltpu.sync_copy(acc_sh.at[pl.ds(r0, T)],
                            out_hbm.at[ch].at[pl.ds(r0, T)])
        plsc.subcore_barrier()


def _pass2(xlc, src, dst, ex, rden):
    f = pl.kernel(
        _pass2_body,
        out_type=jax.ShapeDtypeStruct((8, NPAD, 128), jnp.float32),
        mesh=plsc.VectorSubcoreMesh(
            core_axis_name="c", subcore_axis_name="s",
            num_cores=NC, num_subcores=NS),
        scratch_types=[
            pltpu.VMEM((T,), jnp.int32),
            pltpu.VMEM((T,), jnp.int32),
            pltpu.VMEM((T, 128), jnp.float32),
            pltpu.VMEM((T, 16), jnp.float32),
            pltpu.VMEM((T, 16), jnp.float32),
            pltpu.VMEM((T, 128), jnp.float32),
            pltpu.VMEM_SHARED((NPAD, 128), jnp.float32),
            pltpu.SemaphoreType.DMA,
        ],
    )
    return f(xlc, src, dst, ex, rden)


def _gat_layer(xc_in, eec, src, dst, w_lr, att):
    xall = _proj(xc_in, w_lr, 16)
    xlc, xrc = xall[:8], xall[8:]
    ex, den = _pass1(xlc, xrc, eec, src, dst, att)
    d2 = den.reshape(NC, NPAD // 8, 128)
    rden = _rden(d2[0], d2[1]).reshape(NPAD, 16)
    outc = _pass2(xlc, src, dst, ex, rden)
    return outc


def kernel(x, edge_index, edge_attr, batch, Wl1, Wr1, We1, att1, b1,
           Wl2, Wr2, We2, att2, b2, Wg1, bg1, Wg2, bg2, Wnp, bnp):
    f32 = jnp.float32
    x = x.astype(f32)

    src = jnp.concatenate([edge_index[0].astype(jnp.int32),
                           jnp.arange(N, dtype=jnp.int32),
                           jnp.full((E2P - E2,), DUMMY, jnp.int32)])
    dst = jnp.concatenate([edge_index[1].astype(jnp.int32),
                           jnp.arange(N, dtype=jnp.int32),
                           jnp.full((E2P - E2,), DUMMY, jnp.int32)])

    ea128 = jnp.zeros((EP1, 128), f32).at[:E, :ED].set(edge_attr)
    emean = _colmean(ea128)[0:1, :]

    ea_full = jnp.concatenate([
        ea128[:E],
        jnp.broadcast_to(emean, (N, 128)),
        jnp.zeros((E2P - E2, 128), f32),
    ]).reshape(1, E2P, 128)

    xpad = jnp.zeros((NPAD, D), f32).at[:N].set(x)
    xc0 = xpad.reshape(NPAD, D // 128, 128).transpose(1, 0, 2)

    w12 = jnp.concatenate([Wl1, Wr1], axis=1)
    we1p = jnp.zeros((128, HC), f32).at[:ED].set(We1)
    we2p = jnp.zeros((128, HC), f32).at[:ED].set(We2)

    eec1 = _proj(ea_full, we1p, 8)
    out1 = _gat_layer(xc0, eec1, src, dst, w12, att1)

    b1r = b1.reshape(8, 1, 128)
    yc = _elu(out1, b1r)

    w34 = jnp.concatenate([Wl2, Wr2], axis=1)
    eec2 = _proj(ea_full, we2p, 8)
    out2 = _gat_layer(yc, eec2, src, dst, w34, att2)

    b2r = b2.reshape(8, 1, 128)
    batp = jnp.concatenate(
        [batch.astype(jnp.int32), jnp.full((NPAD - N,), B, jnp.int32)])
    bat2d = jnp.broadcast_to(batp[:, None], (NPAD, 128))
    bg1r = bg1.reshape(1, GH)
    wg2p = jnp.zeros((GH, 128), f32).at[:, :NM * 5].set(Wg2)
    bg2r = jnp.zeros((1, 128), f32).at[0, :NM * 5].set(bg2)
    wnpp = jnp.zeros((HC, 128), f32).at[:, :AD].set(Wnp)
    bnpr = jnp.zeros((1, 128), f32).at[0, :AD].set(bnp)

    gmm_p, npl_p = _pool(out2, b2r, bat2d, Wg1, bg1r, wg2p, bg2r, wnpp, bnpr)
    return (gmm_p[:, :NM * 5], npl_p[:, :AD])


# speedup vs baseline: 2.0288x; 2.0288x over previous
import functools
import jax
import jax.numpy as jnp
from jax import lax
from jax.experimental import pallas as pl
from jax.experimental.pallas import tpu as pltpu
from jax.experimental.pallas import tpu_sc as plsc

N = 10000
E = 160000
D = 256
ED = 16
H = 4
C = 256
HC = 1024
B = 8
NM = 3
AD = 64
GH = 256

NPAD = 10240
DUMMY = NPAD - 1
E2 = E + N
NC = 2
NS = 16
T = 128
E2P = 172032
PW1 = E2P // (NC * NS)
NB1 = PW1 // T
PW2 = E2P // NS
NB2 = PW2 // T
ROWS_PER_TILE = NPAD // NS
QNP = 2560
QNPT = QNP + 128
HNP = 5120
HNPT = HNP + 128


EP1 = 160256


def _colmean_body(x_ref, o_ref):
    @pl.when(pl.program_id(0) == 0)
    def _():
        o_ref[...] = jnp.zeros_like(o_ref)

    o_ref[0:1, :] += jnp.sum(x_ref[...], axis=0, keepdims=True)

    @pl.when(pl.program_id(0) == EP1 // 512 - 1)
    def _():
        o_ref[...] = o_ref[...] * (1.0 / E)


def _colmean(ea128):
    return pl.pallas_call(
        _colmean_body,
        grid=(EP1 // 512,),
        in_specs=[pl.BlockSpec((512, 128), lambda i: (i, 0))],
        out_specs=pl.BlockSpec((8, 128), lambda i: (0, 0)),
        out_shape=jax.ShapeDtypeStruct((8, 128), jnp.float32),
    )(ea128)


def _proj_body(x_ref, w_ref, o_ref):
    @pl.when(pl.program_id(2) == 0)
    def _():
        o_ref[...] = jnp.zeros_like(o_ref)

    o_ref[0] += jnp.dot(x_ref[0], w_ref[...],
                        preferred_element_type=jnp.float32)


def _proj(xc, w, nch):
    kc, mp, _ = xc.shape
    return pl.pallas_call(
        _proj_body,
        grid=(nch, mp // 256, kc),
        in_specs=[
            pl.BlockSpec((1, 256, 128), lambda ct, nt, kt: (kt, nt, 0)),
            pl.BlockSpec((128, 128), lambda ct, nt, kt: (kt, ct)),
        ],
        out_specs=pl.BlockSpec((1, 256, 128), lambda ct, nt, kt: (ct, nt, 0)),
        out_shape=jax.ShapeDtypeStruct((nch, mp, 128), jnp.float32),
    )(xc, w)


def _elu_body(x_ref, b_ref, o_ref):
    t = x_ref[...] + b_ref[...]
    o_ref[...] = jnp.where(t > 0, t, jnp.exp(t) - 1.0)


def _elu(xc, br):
    return pl.pallas_call(
        _elu_body,
        grid=(8, NPAD // 512),
        in_specs=[
            pl.BlockSpec((1, 512, 128), lambda c, n: (c, n, 0)),
            pl.BlockSpec((1, 1, 128), lambda c, n: (c, 0, 0)),
        ],
        out_specs=pl.BlockSpec((1, 512, 128), lambda c, n: (c, n, 0)),
        out_shape=jax.ShapeDtypeStruct((8, NPAD, 128), jnp.float32),
    )(xc, br)


def _rden_body(a_ref, b_ref, o_ref):
    o_ref[...] = 1.0 / (a_ref[...] + b_ref[...])


def _rden(d0, d1):
    return pl.pallas_call(
        _rden_body,
        grid=(NPAD // 512,),
        in_specs=[
            pl.BlockSpec((512, 128), lambda i: (i, 0)),
            pl.BlockSpec((512, 128), lambda i: (i, 0)),
        ],
        out_specs=pl.BlockSpec((512, 128), lambda i: (i, 0)),
        out_shape=jax.ShapeDtypeStruct((NPAD, 128), jnp.float32),
    )(d0, d1)


def _pool_body(y_ref, b2_ref, bat_ref, wg1_ref, bg1_ref, wg2_ref, bg2_ref,
               wnp_ref, bnp_ref, gmm_ref, npl_ref, psum, cnt):
    n = pl.program_id(0)

    @pl.when(n == 0)
    def _():
        psum[...] = jnp.zeros_like(psum)
        cnt[...] = jnp.zeros_like(cnt)

    yb = y_ref[...] + b2_ref[...]
    gidx = lax.broadcasted_iota(jnp.int32, (512, 128), 1)
    ohb = (bat_ref[...] == gidx).astype(jnp.float32)
    psum[...] += lax.dot_general(ohb, yb, (((0,), (1,)), ((), ())))
    cnt[...] += lax.dot_general(ohb, jnp.ones((512, 128), jnp.float32),
                                (((0,), (0,)), ((), ())))

    @pl.when(n == NPAD // 512 - 1)
    def _():
        ps = psum[...].reshape(128, HC)
        c1 = cnt[:, 0:1]
        pooled = ps * (1.0 / jnp.maximum(c1, 1.0))
        hh = jnp.maximum(
            jnp.dot(pooled, wg1_ref[...],
                    preferred_element_type=jnp.float32) + bg1_ref[...], 0.0)
        gmm = jnp.dot(hh, wg2_ref[...],
                      preferred_element_type=jnp.float32) + bg2_ref[...]
        npl = jnp.dot(pooled, wnp_ref[...],
                      preferred_element_type=jnp.float32) + bnp_ref[...]
        gmm_ref[...] = gmm[0:8]
        npl_ref[...] = npl[0:8]


def _pool(y2c, b2r, bat2d, wg1, bg1r, wg2p, bg2r, wnpp, bnpr):
    return pl.pallas_call(
        _pool_body,
        grid=(NPAD // 512,),
        in_specs=[
            pl.BlockSpec((8, 512, 128), lambda n: (0, n, 0)),
            pl.BlockSpec((8, 1, 128), lambda n: (0, 0, 0)),
            pl.BlockSpec((512, 128), lambda n: (n, 0)),
            pl.BlockSpec((HC, GH), lambda n: (0, 0)),
            pl.BlockSpec((1, GH), lambda n: (0, 0)),
            pl.BlockSpec((GH, 128), lambda n: (0, 0)),
            pl.BlockSpec((1, 128), lambda n: (0, 0)),
            pl.BlockSpec((HC, 128), lambda n: (0, 0)),
            pl.BlockSpec((1, 128), lambda n: (0, 0)),
        ],
        out_specs=[
            pl.BlockSpec((8, 128), lambda n: (0, 0)),
            pl.BlockSpec((8, 128), lambda n: (0, 0)),
        ],
        out_shape=[
            jax.ShapeDtypeStruct((8, 128), jnp.float32),
            jax.ShapeDtypeStruct((8, 128), jnp.float32),
        ],
        scratch_shapes=[
            pltpu.VMEM((128, 8, 128), jnp.float32),
            pltpu.VMEM((128, 128), jnp.float32),
        ],
    )(y2c, b2r, bat2d, wg1, bg1r, wg2p, bg2r, wnpp, bnpr)


def _pass1_body(xlc, xrc, eec, srch, dsth, atth, ex_out,
                srcv, dstv, bufa, bufb, bufe, exb, alph, attv,
                sema, semb):
    ci = lax.axis_index("c")
    sid = lax.axis_index("s")
    wid = sid * NC + ci
    iota16 = lax.broadcasted_iota(jnp.int32, (16,), 0)
    zeros16 = jnp.zeros((16,), jnp.float32)

    def zrow(r, _):
        for v in range(8):
            exb[r, pl.ds(16 * v, 16)] = zeros16
        return 0

    lax.fori_loop(0, T, zrow, 0)
    pltpu.sync_copy(atth, attv)

    def batch(g, _):
        e0 = wid * PW1 + g * T
        pltpu.sync_copy(srch.at[pl.ds(e0, T)], srcv)
        pltpu.sync_copy(dsth.at[pl.ds(e0, T)], dstv)

        def za(e, _):
            alph[e, :] = zeros16
            return 0

        lax.fori_loop(0, T, za, 0)

        for ch in range(8):
            h = ch // 2
            cpa = pltpu.async_copy(xlc.at[ch].at[srcv], bufa, sema)
            cpb = pltpu.async_copy(xrc.at[ch].at[dstv], bufb, semb)
            pltpu.sync_copy(eec.at[ch].at[pl.ds(e0, T)], bufe)
            cpa.wait()
            cpb.wait()
            attc = [attv[h, pl.ds((ch % 2) * 128 + 16 * v, 16)]
                    for v in range(8)]
            hmask = iota16 == h

            def ebody(e, _):
                acc = zeros16
                for v in range(8):
                    sl = pl.ds(16 * v, 16)
                    t = bufa[e, sl] + bufb[e, sl] + bufe[e, sl]
                    m = jnp.maximum(t, 0.2 * t)
                    acc = acc + m * attc[v]
                s = jnp.sum(acc)
                alph[e, :] = alph[e, :] + jnp.where(hmask, s, 0.0)
                return 0

            lax.fori_loop(0, T, ebody, 0)

        def xbody(e, _):
            exb[e, pl.ds(0, 16)] = jnp.exp(jnp.minimum(alph[e, :], 60.0))
            return 0

        lax.fori_loop(0, T, xbody, 0)
        pltpu.sync_copy(exb, ex_out.at[pl.ds(e0, T)])
        return 0

    lax.fori_loop(0, NB1, batch, 0)


def _pass1(xlc, xrc, eec, src, dst, att):
    f = pl.kernel(
        _pass1_body,
        out_type=jax.ShapeDtypeStruct((E2P, 128), jnp.float32),
        mesh=plsc.VectorSubcoreMesh(
            core_axis_name="c", subcore_axis_name="s",
            num_cores=NC, num_subcores=NS),
        compiler_params=pltpu.CompilerParams(needs_layout_passes=False),
        scratch_types=[
            pltpu.VMEM((T,), jnp.int32),
            pltpu.VMEM((T,), jnp.int32),
            pltpu.VMEM((T, 128), jnp.float32),
            pltpu.VMEM((T, 128), jnp.float32),
            pltpu.VMEM((T, 128), jnp.float32),
            pltpu.VMEM((T, 128), jnp.float32),
            pltpu.VMEM((T, 16), jnp.float32),
            pltpu.VMEM((H, C), jnp.float32),
            pltpu.SemaphoreType.DMA,
            pltpu.SemaphoreType.DMA,
        ],
    )
    return f(xlc, xrc, eec, src, dst, att)


def _den_body(dsth, exh, den_out, dstv, dstl, exb, zb, den_sh):
    ci = lax.axis_index("c")
    sid = lax.axis_index("s")
    wid = sid * NC + ci
    rpt = QNPT // NS

    def zrow(r, _):
        for v in range(8):
            zb[r, pl.ds(16 * v, 16)] = jnp.zeros((16,), jnp.float32)
        return 0

    lax.fori_loop(0, T, zrow, 0)

    for q in range(4):
        base = q * QNP
        r0 = sid * rpt
        pltpu.sync_copy(zb, den_sh.at[pl.ds(r0, T)])
        pltpu.sync_copy(zb.at[0:40], den_sh.at[pl.ds(r0 + T, 40)])
        plsc.subcore_barrier()

        def batch(g, _):
            e0 = wid * PW1 + g * T
            pltpu.sync_copy(dsth.at[pl.ds(e0, T)], dstv)
            pltpu.sync_copy(exh.at[pl.ds(e0, T)], exb)

            def fixd(i, _):
                sl = pl.ds(16 * i, 16)
                d = dstv[sl] - base
                ok = (d >= 0) & (d < QNP)
                dstl[sl] = jnp.where(ok, d, QNP)
                return 0

            lax.fori_loop(0, 8, fixd, 0)
            pltpu.sync_copy(exb, den_sh.at[dstl], add=True)
            return 0

        lax.fori_loop(0, NB1, batch, 0)
        plsc.subcore_barrier()
        crpt = QNP // NS
        c0 = sid * crpt
        pltpu.sync_copy(den_sh.at[pl.ds(c0, 128)],
                        den_out.at[ci].at[q].at[pl.ds(c0, 128)])
        pltpu.sync_copy(den_sh.at[pl.ds(c0 + 128, 32)],
                        den_out.at[ci].at[q].at[pl.ds(c0 + 128, 32)])
        plsc.subcore_barrier()


def _den(dst, ex):
    f = pl.kernel(
        _den_body,
        out_type=jax.ShapeDtypeStruct((NC, 4, QNP, 128), jnp.float32),
        mesh=plsc.VectorSubcoreMesh(
            core_axis_name="c", subcore_axis_name="s",
            num_cores=NC, num_subcores=NS),
        compiler_params=pltpu.CompilerParams(needs_layout_passes=False),
        scratch_types=[
            pltpu.VMEM((T,), jnp.int32),
            pltpu.VMEM((T,), jnp.int32),
            pltpu.VMEM((T, 128), jnp.float32),
            pltpu.VMEM((T, 128), jnp.float32),
            pltpu.VMEM_SHARED((QNPT, 128), jnp.float32),
        ],
    )
    return f(dst, ex)


def _pass2_body(xlc, srch, dsth, exh, rdenh, out_hbm,
                srcv, dstv, dstl, buf, exb, rdb, zb, acc_sh, sem):
    ci = lax.axis_index("c")
    sid = lax.axis_index("s")
    iota16 = lax.broadcasted_iota(jnp.int32, (16,), 0)
    zeros16 = jnp.zeros((16,), jnp.float32)

    def zrow(r, _):
        for v in range(8):
            zb[r, pl.ds(16 * v, 16)] = zeros16
        return 0

    lax.fori_loop(0, T, zrow, 0)

    for cc in range(4):
        ch = ci * 4 + cc
        h = ch // 2
        hmask = iota16 == h
        for half in range(2):
            base = half * HNP
            r0 = sid * (HNPT // NS)
            pltpu.sync_copy(zb, acc_sh.at[pl.ds(r0, T)])
            pltpu.sync_copy(zb, acc_sh.at[pl.ds(r0 + T, T)])
            pltpu.sync_copy(zb.at[0:72], acc_sh.at[pl.ds(r0 + 2 * T, 72)])
            plsc.subcore_barrier()

            def batch(g, _):
                e0 = sid * PW2 + g * T
                pltpu.sync_copy(srch.at[pl.ds(e0, T)], srcv)
                pltpu.sync_copy(dsth.at[pl.ds(e0, T)], dstv)
                pltpu.sync_copy(exh.at[pl.ds(e0, T)], exb)
                pltpu.async_copy(rdenh.at[dstv], rdb, sem).wait()
                pltpu.async_copy(xlc.at[ch].at[srcv], buf, sem).wait()

                def fixd(i, _):
                    sl = pl.ds(16 * i, 16)
                    d = dstv[sl] - base
                    ok = (d >= 0) & (d < HNP)
                    dstl[sl] = jnp.where(ok, d, HNP)
                    return 0

                lax.fori_loop(0, 8, fixd, 0)

                def ebody(e, _):
                    p = exb[e, pl.ds(0, 16)] * rdb[e, pl.ds(0, 16)]
                    a = jnp.sum(jnp.where(hmask, p, 0.0))
                    for v in range(8):
                        sl = pl.ds(16 * v, 16)
                        buf[e, sl] = buf[e, sl] * a
                    return 0

                lax.fori_loop(0, T, ebody, 0)
                pltpu.sync_copy(buf, acc_sh.at[dstl], add=True)
                return 0

            lax.fori_loop(0, NB2, batch, 0)
            plsc.subcore_barrier()
            c0 = sid * (HNP // NS)
            pltpu.sync_copy(acc_sh.at[pl.ds(c0, T)],
                            out_hbm.at[ch].at[pl.ds(base + c0, T)])
            pltpu.sync_copy(acc_sh.at[pl.ds(c0 + T, T)],
                            out_hbm.at[ch].at[pl.ds(base + c0 + T, T)])
            pltpu.sync_copy(acc_sh.at[pl.ds(c0 + 2 * T, 64)],
                            out_hbm.at[ch].at[pl.ds(base + c0 + 2 * T, 64)])
            plsc.subcore_barrier()


def _pass2(xlc, src, dst, ex, rden):
    f = pl.kernel(
        _pass2_body,
        out_type=jax.ShapeDtypeStruct((8, NPAD, 128), jnp.float32),
        mesh=plsc.VectorSubcoreMesh(
            core_axis_name="c", subcore_axis_name="s",
            num_cores=NC, num_subcores=NS),
        compiler_params=pltpu.CompilerParams(needs_layout_passes=False),
        scratch_types=[
            pltpu.VMEM((T,), jnp.int32),
            pltpu.VMEM((T,), jnp.int32),
            pltpu.VMEM((T,), jnp.int32),
            pltpu.VMEM((T, 128), jnp.float32),
            pltpu.VMEM((T, 128), jnp.float32),
            pltpu.VMEM((T, 128), jnp.float32),
            pltpu.VMEM((T, 128), jnp.float32),
            pltpu.VMEM_SHARED((HNPT, 128), jnp.float32),
            pltpu.SemaphoreType.DMA,
        ],
    )
    return f(xlc, src, dst, ex, rden)


def _gat_layer(xc_in, eec, src, dst, w_lr, att):
    xall = _proj(xc_in, w_lr, 16)
    xlc, xrc = xall[:8], xall[8:]
    ex = _pass1(xlc, xrc, eec, src, dst, att)
    den = _den(dst, ex)
    d0 = den[0].reshape(NPAD, 128)
    d1 = den[1].reshape(NPAD, 128)
    rden = _rden(d0, d1)
    outc = _pass2(xlc, src, dst, ex, rden)
    return outc


def kernel(x, edge_index, edge_attr, batch, Wl1, Wr1, We1, att1, b1,
           Wl2, Wr2, We2, att2, b2, Wg1, bg1, Wg2, bg2, Wnp, bnp):
    f32 = jnp.float32
    x = x.astype(f32)

    src = jnp.concatenate([edge_index[0].astype(jnp.int32),
                           jnp.arange(N, dtype=jnp.int32),
                           jnp.full((E2P - E2,), DUMMY, jnp.int32)])
    dst = jnp.concatenate([edge_index[1].astype(jnp.int32),
                           jnp.arange(N, dtype=jnp.int32),
                           jnp.full((E2P - E2,), DUMMY, jnp.int32)])

    ea128 = jnp.zeros((EP1, 128), f32).at[:E, :ED].set(edge_attr)
    emean = _colmean(ea128)[0:1, :]

    ea_full = jnp.concatenate([
        ea128[:E],
        jnp.broadcast_to(emean, (N, 128)),
        jnp.zeros((E2P - E2, 128), f32),
    ]).reshape(1, E2P, 128)

    xpad = jnp.zeros((NPAD, D), f32).at[:N].set(x)
    xc0 = xpad.reshape(NPAD, D // 128, 128).transpose(1, 0, 2)

    w12 = jnp.concatenate([Wl1, Wr1], axis=1)
    we1p = jnp.zeros((128, HC), f32).at[:ED].set(We1)
    we2p = jnp.zeros((128, HC), f32).at[:ED].set(We2)

    eec1 = _proj(ea_full, we1p, 8)
    out1 = _gat_layer(xc0, eec1, src, dst, w12, att1)

    b1r = b1.reshape(8, 1, 128)
    yc = _elu(out1, b1r)

    w34 = jnp.concatenate([Wl2, Wr2], axis=1)
    eec2 = _proj(ea_full, we2p, 8)
    out2 = _gat_layer(yc, eec2, src, dst, w34, att2)

    b2r = b2.reshape(8, 1, 128)
    batp = jnp.concatenate(
        [batch.astype(jnp.int32), jnp.full((NPAD - N,), B, jnp.int32)])
    bat2d = jnp.broadcast_to(batp[:, None], (NPAD, 128))
    bg1r = bg1.reshape(1, GH)
    wg2p = jnp.zeros((GH, 128), f32).at[:, :NM * 5].set(Wg2)
    bg2r = jnp.zeros((1, 128), f32).at[0, :NM * 5].set(bg2)
    wnpp = jnp.zeros((HC, 128), f32).at[:, :AD].set(Wnp)
    bnpr = jnp.zeros((1, 128), f32).at[0, :AD].set(bnp)

    gmm_p, npl_p = _pool(out2, b2r, bat2d, Wg1, bg1r, wg2p, bg2r, wnpp, bnpr)
    return (gmm_p[:, :NM * 5], npl_p[:, :AD])


# speedup vs baseline: 2.7560x; 1.3585x over previous
import functools
import jax
import jax.numpy as jnp
from jax import lax
from jax.experimental import pallas as pl
from jax.experimental.pallas import tpu as pltpu
from jax.experimental.pallas import tpu_sc as plsc

N = 10000
E = 160000
D = 256
ED = 16
H = 4
C = 256
HC = 1024
B = 8
NM = 3
AD = 64
GH = 256

NPAD = 10240
DUMMY = NPAD - 1
E2 = E + N
NC = 2
NS = 16
T = 128
E2P = 172032
PW1 = E2P // (NC * NS)
NB1 = PW1 // T
PW2 = E2P // NS
NB2 = PW2 // T
ROWS_PER_TILE = NPAD // NS
QNP = 2560
QNPT = QNP + 128
HNP = 5120
HNPT = HNP + 128


EP1 = 163840


def _colmean_body(x_ref, o_ref):
    @pl.when(pl.program_id(0) == 0)
    def _():
        o_ref[...] = jnp.zeros_like(o_ref)

    o_ref[0:1, :] += jnp.sum(x_ref[...], axis=0, keepdims=True)

    @pl.when(pl.program_id(0) == EP1 // 4096 - 1)
    def _():
        o_ref[...] = o_ref[...] * (1.0 / E)


def _colmean(ea128):
    return pl.pallas_call(
        _colmean_body,
        grid=(EP1 // 4096,),
        in_specs=[pl.BlockSpec((4096, 128), lambda i: (i, 0))],
        out_specs=pl.BlockSpec((8, 128), lambda i: (0, 0)),
        out_shape=jax.ShapeDtypeStruct((8, 128), jnp.float32),
    )(ea128)


def _proj_body(x_ref, w_ref, o_ref):
    nch = o_ref.shape[0]
    res = jnp.dot(x_ref[...], w_ref[...], preferred_element_type=jnp.float32)
    for c in range(nch):
        o_ref[c] = res[:, 128 * c:128 * (c + 1)]


def _proj(x, w, nch, bm):
    mp, k = x.shape
    return pl.pallas_call(
        _proj_body,
        grid=(mp // bm,),
        in_specs=[
            pl.BlockSpec((bm, k), lambda nt: (nt, 0)),
            pl.BlockSpec((k, nch * 128), lambda nt: (0, 0)),
        ],
        out_specs=pl.BlockSpec((nch, bm, 128), lambda nt: (0, nt, 0)),
        out_shape=jax.ShapeDtypeStruct((nch, mp, 128), jnp.float32),
    )(x, w)


def _elu_body(x_ref, b_ref, o_ref):
    t = x_ref[0] + b_ref[0]
    o_ref[...] = jnp.where(t > 0, t, jnp.exp(t) - 1.0)


def _elu(xc, br):
    return pl.pallas_call(
        _elu_body,
        grid=(8, NPAD // 2048),
        in_specs=[
            pl.BlockSpec((1, 2048, 128), lambda c, n: (c, n, 0)),
            pl.BlockSpec((1, 1, 128), lambda c, n: (c, 0, 0)),
        ],
        out_specs=pl.BlockSpec((2048, 128), lambda c, n: (n, c)),
        out_shape=jax.ShapeDtypeStruct((NPAD, HC), jnp.float32),
    )(xc, br)


def _rden_body(a_ref, b_ref, o_ref):
    o_ref[...] = 1.0 / (a_ref[...] + b_ref[...])


def _rden(d0, d1):
    return pl.pallas_call(
        _rden_body,
        grid=(NPAD // 512,),
        in_specs=[
            pl.BlockSpec((512, 128), lambda i: (i, 0)),
            pl.BlockSpec((512, 128), lambda i: (i, 0)),
        ],
        out_specs=pl.BlockSpec((512, 128), lambda i: (i, 0)),
        out_shape=jax.ShapeDtypeStruct((NPAD, 128), jnp.float32),
    )(d0, d1)


def _pool_body(y_ref, b2_ref, bat_ref, wg1_ref, bg1_ref, wg2_ref, bg2_ref,
               wnp_ref, bnp_ref, gmm_ref, npl_ref, psum, cnt):
    n = pl.program_id(0)

    @pl.when(n == 0)
    def _():
        psum[...] = jnp.zeros_like(psum)
        cnt[...] = jnp.zeros_like(cnt)

    yb = y_ref[...] + b2_ref[...]
    gidx = lax.broadcasted_iota(jnp.int32, (512, 128), 1)
    ohb = (bat_ref[...] == gidx).astype(jnp.float32)
    psum[...] += lax.dot_general(ohb, yb, (((0,), (1,)), ((), ())))
    cnt[...] += lax.dot_general(ohb, jnp.ones((512, 128), jnp.float32),
                                (((0,), (0,)), ((), ())))

    @pl.when(n == NPAD // 512 - 1)
    def _():
        ps = psum[...].reshape(128, HC)
        c1 = cnt[:, 0:1]
        pooled = ps * (1.0 / jnp.maximum(c1, 1.0))
        hh = jnp.maximum(
            jnp.dot(pooled, wg1_ref[...],
                    preferred_element_type=jnp.float32) + bg1_ref[...], 0.0)
        gmm = jnp.dot(hh, wg2_ref[...],
                      preferred_element_type=jnp.float32) + bg2_ref[...]
        npl = jnp.dot(pooled, wnp_ref[...],
                      preferred_element_type=jnp.float32) + bnp_ref[...]
        gmm_ref[...] = gmm[0:8]
        npl_ref[...] = npl[0:8]


def _pool(y2c, b2r, bat2d, wg1, bg1r, wg2p, bg2r, wnpp, bnpr):
    return pl.pallas_call(
        _pool_body,
        grid=(NPAD // 512,),
        in_specs=[
            pl.BlockSpec((8, 512, 128), lambda n: (0, n, 0)),
            pl.BlockSpec((8, 1, 128), lambda n: (0, 0, 0)),
            pl.BlockSpec((512, 128), lambda n: (n, 0)),
            pl.BlockSpec((HC, GH), lambda n: (0, 0)),
            pl.BlockSpec((1, GH), lambda n: (0, 0)),
            pl.BlockSpec((GH, 128), lambda n: (0, 0)),
            pl.BlockSpec((1, 128), lambda n: (0, 0)),
            pl.BlockSpec((HC, 128), lambda n: (0, 0)),
            pl.BlockSpec((1, 128), lambda n: (0, 0)),
        ],
        out_specs=[
            pl.BlockSpec((8, 128), lambda n: (0, 0)),
            pl.BlockSpec((8, 128), lambda n: (0, 0)),
        ],
        out_shape=[
            jax.ShapeDtypeStruct((8, 128), jnp.float32),
            jax.ShapeDtypeStruct((8, 128), jnp.float32),
        ],
        scratch_shapes=[
            pltpu.VMEM((128, 8, 128), jnp.float32),
            pltpu.VMEM((128, 128), jnp.float32),
        ],
    )(y2c, b2r, bat2d, wg1, bg1r, wg2p, bg2r, wnpp, bnpr)


def _pass1_body(xlc, xrc, eec, srch, dsth, atth, ex_out,
                srcv, dstv, bufa, bufb, bufe, exb, alph, attv,
                sema, semb):
    ci = lax.axis_index("c")
    sid = lax.axis_index("s")
    wid = sid * NC + ci
    iota16 = lax.broadcasted_iota(jnp.int32, (16,), 0)
    zeros16 = jnp.zeros((16,), jnp.float32)

    def zrow(r, _):
        for v in range(8):
            exb[r, pl.ds(16 * v, 16)] = zeros16
        return 0

    lax.fori_loop(0, T, zrow, 0)
    pltpu.sync_copy(atth, attv)

    def batch(g, _):
        e0 = wid * PW1 + g * T
        pltpu.sync_copy(srch.at[pl.ds(e0, T)], srcv)
        pltpu.sync_copy(dsth.at[pl.ds(e0, T)], dstv)

        def za(e, _):
            alph[e, :] = zeros16
            return 0

        lax.fori_loop(0, T, za, 0)

        for ch in range(8):
            h = ch // 2
            cpa = pltpu.async_copy(xlc.at[ch].at[srcv], bufa, sema)
            cpb = pltpu.async_copy(xrc.at[ch].at[dstv], bufb, semb)
            pltpu.sync_copy(eec.at[ch].at[pl.ds(e0, T)], bufe)
            cpa.wait()
            cpb.wait()
            attc = [attv[h, pl.ds((ch % 2) * 128 + 16 * v, 16)]
                    for v in range(8)]
            hmask = iota16 == h

            def ebody(e, _):
                acc = zeros16
                for v in range(8):
                    sl = pl.ds(16 * v, 16)
                    t = bufa[e, sl] + bufb[e, sl] + bufe[e, sl]
                    m = jnp.maximum(t, 0.2 * t)
                    acc = acc + m * attc[v]
                s = jnp.sum(acc)
                alph[e, :] = alph[e, :] + jnp.where(hmask, s, 0.0)
                return 0

            lax.fori_loop(0, T, ebody, 0)

        def xbody(e, _):
            exb[e, pl.ds(0, 16)] = jnp.exp(jnp.minimum(alph[e, :], 60.0))
            return 0

        lax.fori_loop(0, T, xbody, 0)
        pltpu.sync_copy(exb, ex_out.at[pl.ds(e0, T)])
        return 0

    lax.fori_loop(0, NB1, batch, 0)


def _pass1(xlc, xrc, eec, src, dst, att):
    f = pl.kernel(
        _pass1_body,
        out_type=jax.ShapeDtypeStruct((E2P, 128), jnp.float32),
        mesh=plsc.VectorSubcoreMesh(
            core_axis_name="c", subcore_axis_name="s",
            num_cores=NC, num_subcores=NS),
        compiler_params=pltpu.CompilerParams(needs_layout_passes=False),
        scratch_types=[
            pltpu.VMEM((T,), jnp.int32),
            pltpu.VMEM((T,), jnp.int32),
            pltpu.VMEM((T, 128), jnp.float32),
            pltpu.VMEM((T, 128), jnp.float32),
            pltpu.VMEM((T, 128), jnp.float32),
            pltpu.VMEM((T, 128), jnp.float32),
            pltpu.VMEM((T, 16), jnp.float32),
            pltpu.VMEM((H, C), jnp.float32),
            pltpu.SemaphoreType.DMA,
            pltpu.SemaphoreType.DMA,
        ],
    )
    return f(xlc, xrc, eec, src, dst, att)


def _den_body(dsth, exh, den_out, dstv, dstl, exb, zb, den_sh):
    ci = lax.axis_index("c")
    sid = lax.axis_index("s")
    wid = sid * NC + ci
    rpt = QNPT // NS

    def zrow(r, _):
        for v in range(8):
            zb[r, pl.ds(16 * v, 16)] = jnp.zeros((16,), jnp.float32)
        return 0

    lax.fori_loop(0, T, zrow, 0)

    for q in range(4):
        base = q * QNP
        r0 = sid * rpt
        pltpu.sync_copy(zb, den_sh.at[pl.ds(r0, T)])
        pltpu.sync_copy(zb.at[0:40], den_sh.at[pl.ds(r0 + T, 40)])
        plsc.subcore_barrier()

        def batch(g, _):
            e0 = wid * PW1 + g * T
            pltpu.sync_copy(dsth.at[pl.ds(e0, T)], dstv)
            pltpu.sync_copy(exh.at[pl.ds(e0, T)], exb)

            def fixd(i, _):
                sl = pl.ds(16 * i, 16)
                d = dstv[sl] - base
                ok = (d >= 0) & (d < QNP)
                dstl[sl] = jnp.where(ok, d, QNP)
                return 0

            lax.fori_loop(0, 8, fixd, 0)
            pltpu.sync_copy(exb, den_sh.at[dstl], add=True)
            return 0

        lax.fori_loop(0, NB1, batch, 0)
        plsc.subcore_barrier()
        crpt = QNP // NS
        c0 = sid * crpt
        pltpu.sync_copy(den_sh.at[pl.ds(c0, 128)],
                        den_out.at[ci].at[q].at[pl.ds(c0, 128)])
        pltpu.sync_copy(den_sh.at[pl.ds(c0 + 128, 32)],
                        den_out.at[ci].at[q].at[pl.ds(c0 + 128, 32)])
        plsc.subcore_barrier()


def _den(dst, ex):
    f = pl.kernel(
        _den_body,
        out_type=jax.ShapeDtypeStruct((NC, 4, QNP, 128), jnp.float32),
        mesh=plsc.VectorSubcoreMesh(
            core_axis_name="c", subcore_axis_name="s",
            num_cores=NC, num_subcores=NS),
        compiler_params=pltpu.CompilerParams(needs_layout_passes=False),
        scratch_types=[
            pltpu.VMEM((T,), jnp.int32),
            pltpu.VMEM((T,), jnp.int32),
            pltpu.VMEM((T, 128), jnp.float32),
            pltpu.VMEM((T, 128), jnp.float32),
            pltpu.VMEM_SHARED((QNPT, 128), jnp.float32),
        ],
    )
    return f(dst, ex)


def _pass2_body(xlc, srch, dsth, exh, rdenh, out_hbm,
                srcv, dstv, dstl, buf, exb, rdb, zb, acc_sh, sem):
    ci = lax.axis_index("c")
    sid = lax.axis_index("s")
    iota16 = lax.broadcasted_iota(jnp.int32, (16,), 0)
    zeros16 = jnp.zeros((16,), jnp.float32)

    def zrow(r, _):
        for v in range(8):
            zb[r, pl.ds(16 * v, 16)] = zeros16
        return 0

    lax.fori_loop(0, T, zrow, 0)

    for cc in range(4):
        ch = ci * 4 + cc
        h = ch // 2
        hmask = iota16 == h
        for half in range(2):
            base = half * HNP
            r0 = sid * (HNPT // NS)
            pltpu.sync_copy(zb, acc_sh.at[pl.ds(r0, T)])
            pltpu.sync_copy(zb, acc_sh.at[pl.ds(r0 + T, T)])
            pltpu.sync_copy(zb.at[0:72], acc_sh.at[pl.ds(r0 + 2 * T, 72)])
            plsc.subcore_barrier()

            def batch(g, _):
                e0 = sid * PW2 + g * T
                pltpu.sync_copy(srch.at[pl.ds(e0, T)], srcv)
                pltpu.sync_copy(dsth.at[pl.ds(e0, T)], dstv)
                pltpu.sync_copy(exh.at[pl.ds(e0, T)], exb)
                pltpu.async_copy(rdenh.at[dstv], rdb, sem).wait()
                pltpu.async_copy(xlc.at[ch].at[srcv], buf, sem).wait()

                def fixd(i, _):
                    sl = pl.ds(16 * i, 16)
                    d = dstv[sl] - base
                    ok = (d >= 0) & (d < HNP)
                    dstl[sl] = jnp.where(ok, d, HNP)
                    return 0

                lax.fori_loop(0, 8, fixd, 0)

                def ebody(e, _):
                    p = exb[e, pl.ds(0, 16)] * rdb[e, pl.ds(0, 16)]
                    a = jnp.sum(jnp.where(hmask, p, 0.0))
                    for v in range(8):
                        sl = pl.ds(16 * v, 16)
                        buf[e, sl] = buf[e, sl] * a
                    return 0

                lax.fori_loop(0, T, ebody, 0)
                pltpu.sync_copy(buf, acc_sh.at[dstl], add=True)
                return 0

            lax.fori_loop(0, NB2, batch, 0)
            plsc.subcore_barrier()
            c0 = sid * (HNP // NS)
            pltpu.sync_copy(acc_sh.at[pl.ds(c0, T)],
                            out_hbm.at[ch].at[pl.ds(base + c0, T)])
            pltpu.sync_copy(acc_sh.at[pl.ds(c0 + T, T)],
                            out_hbm.at[ch].at[pl.ds(base + c0 + T, T)])
            pltpu.sync_copy(acc_sh.at[pl.ds(c0 + 2 * T, 64)],
                            out_hbm.at[ch].at[pl.ds(base + c0 + 2 * T, 64)])
            plsc.subcore_barrier()


def _pass2(xlc, src, dst, ex, rden):
    f = pl.kernel(
        _pass2_body,
        out_type=jax.ShapeDtypeStruct((8, NPAD, 128), jnp.float32),
        mesh=plsc.VectorSubcoreMesh(
            core_axis_name="c", subcore_axis_name="s",
            num_cores=NC, num_subcores=NS),
        compiler_params=pltpu.CompilerParams(needs_layout_passes=False),
        scratch_types=[
            pltpu.VMEM((T,), jnp.int32),
            pltpu.VMEM((T,), jnp.int32),
            pltpu.VMEM((T,), jnp.int32),
            pltpu.VMEM((T, 128), jnp.float32),
            pltpu.VMEM((T, 128), jnp.float32),
            pltpu.VMEM((T, 128), jnp.float32),
            pltpu.VMEM((T, 128), jnp.float32),
            pltpu.VMEM_SHARED((HNPT, 128), jnp.float32),
            pltpu.SemaphoreType.DMA,
        ],
    )
    return f(xlc, src, dst, ex, rden)


def _gat_layer(x_in, eec, src, dst, w_lr, att):
    xall = _proj(x_in, w_lr, 16, 1024)
    xlc, xrc = xall[:8], xall[8:]
    ex = _pass1(xlc, xrc, eec, src, dst, att)
    den = _den(dst, ex)
    d0 = den[0].reshape(NPAD, 128)
    d1 = den[1].reshape(NPAD, 128)
    rden = _rden(d0, d1)
    outc = _pass2(xlc, src, dst, ex, rden)
    return outc


def kernel(x, edge_index, edge_attr, batch, Wl1, Wr1, We1, att1, b1,
           Wl2, Wr2, We2, att2, b2, Wg1, bg1, Wg2, bg2, Wnp, bnp):
    f32 = jnp.float32
    x = x.astype(f32)

    src = jnp.concatenate([edge_index[0].astype(jnp.int32),
                           jnp.arange(N, dtype=jnp.int32),
                           jnp.full((E2P - E2,), DUMMY, jnp.int32)])
    dst = jnp.concatenate([edge_index[1].astype(jnp.int32),
                           jnp.arange(N, dtype=jnp.int32),
                           jnp.full((E2P - E2,), DUMMY, jnp.int32)])

    ea128 = jnp.zeros((EP1, 128), f32).at[:E, :ED].set(edge_attr)
    emean = _colmean(ea128)[0:1, :]

    ea_full = jnp.concatenate([
        ea128[:E],
        jnp.broadcast_to(emean, (N, 128)),
        jnp.zeros((E2P - E2, 128), f32),
    ])

    xpad = jnp.zeros((NPAD, D), f32).at[:N].set(x)

    w12 = jnp.concatenate([Wl1, Wr1], axis=1)
    we1p = jnp.zeros((128, HC), f32).at[:ED].set(We1)
    we2p = jnp.zeros((128, HC), f32).at[:ED].set(We2)

    eec1 = _proj(ea_full, we1p, 8, 2048)
    out1 = _gat_layer(xpad, eec1, src, dst, w12, att1)

    b1r = b1.reshape(8, 1, 128)
    yc = _elu(out1, b1r)

    w34 = jnp.concatenate([Wl2, Wr2], axis=1)
    eec2 = _proj(ea_full, we2p, 8, 2048)
    out2 = _gat_layer(yc, eec2, src, dst, w34, att2)

    b2r = b2.reshape(8, 1, 128)
    batp = jnp.concatenate(
        [batch.astype(jnp.int32), jnp.full((NPAD - N,), B, jnp.int32)])
    bat2d = jnp.broadcast_to(batp[:, None], (NPAD, 128))
    bg1r = bg1.reshape(1, GH)
    wg2p = jnp.zeros((GH, 128), f32).at[:, :NM * 5].set(Wg2)
    bg2r = jnp.zeros((1, 128), f32).at[0, :NM * 5].set(bg2)
    wnpp = jnp.zeros((HC, 128), f32).at[:, :AD].set(Wnp)
    bnpr = jnp.zeros((1, 128), f32).at[0, :AD].set(bnp)

    gmm_p, npl_p = _pool(out2, b2r, bat2d, Wg1, bg1r, wg2p, bg2r, wnpp, bnpr)
    return (gmm_p[:, :NM * 5], npl_p[:, :AD])


# speedup vs baseline: 2.7563x; 1.0001x over previous
import jax
import jax.numpy as jnp
from jax import lax
from jax.experimental import pallas as pl
from jax.experimental.pallas import tpu as pltpu
from jax.experimental.pallas import tpu_sc as plsc

N = 10000
E = 160000
D = 256
ED = 16
H = 4
C = 256
HC = 1024
B = 8
NM = 3
AD = 64
GH = 256

NPAD = 10240
DUMMY = NPAD - 1
E2 = E + N
NC = 2
NS = 16
T = 128
E2P = 172032
PW1 = E2P // (NC * NS)
NB1 = PW1 // T
PW2 = E2P // NS
NB2 = PW2 // T
ROWS_PER_TILE = NPAD // NS
QNP = 2560
QNPT = QNP + 128
HNP = 5120
HNPT = HNP + 128


EP1 = 163840


def _colmean_body(x_ref, o_ref):
    @pl.when(pl.program_id(0) == 0)
    def _():
        o_ref[...] = jnp.zeros_like(o_ref)

    o_ref[0:1, :] += jnp.sum(x_ref[...], axis=0, keepdims=True)

    @pl.when(pl.program_id(0) == EP1 // 4096 - 1)
    def _():
        o_ref[...] = o_ref[...] * (1.0 / E)


def _colmean(ea128):
    return pl.pallas_call(
        _colmean_body,
        grid=(EP1 // 4096,),
        in_specs=[pl.BlockSpec((4096, 128), lambda i: (i, 0))],
        out_specs=pl.BlockSpec((8, 128), lambda i: (0, 0)),
        out_shape=jax.ShapeDtypeStruct((8, 128), jnp.float32),
    )(ea128)


def _proj_body(x_ref, w_ref, o_ref):
    nch = o_ref.shape[0]
    res = jnp.dot(x_ref[...], w_ref[...], preferred_element_type=jnp.float32)
    for c in range(nch):
        o_ref[c] = res[:, 128 * c:128 * (c + 1)]


def _proj(x, w, nch, bm):
    mp, k = x.shape
    return pl.pallas_call(
        _proj_body,
        grid=(mp // bm,),
        in_specs=[
            pl.BlockSpec((bm, k), lambda nt: (nt, 0)),
            pl.BlockSpec((k, nch * 128), lambda nt: (0, 0)),
        ],
        out_specs=pl.BlockSpec((nch, bm, 128), lambda nt: (0, nt, 0)),
        out_shape=jax.ShapeDtypeStruct((nch, mp, 128), jnp.float32),
    )(x, w)


def _elu_body(x_ref, b_ref, o_ref):
    t = x_ref[0] + b_ref[0]
    o_ref[...] = jnp.where(t > 0, t, jnp.exp(t) - 1.0)


def _elu(xc, br):
    return pl.pallas_call(
        _elu_body,
        grid=(8, NPAD // 2048),
        in_specs=[
            pl.BlockSpec((1, 2048, 128), lambda c, n: (c, n, 0)),
            pl.BlockSpec((1, 1, 128), lambda c, n: (c, 0, 0)),
        ],
        out_specs=pl.BlockSpec((2048, 128), lambda c, n: (n, c)),
        out_shape=jax.ShapeDtypeStruct((NPAD, HC), jnp.float32),
    )(xc, br)


def _rden_body(a_ref, b_ref, o_ref):
    o_ref[...] = 1.0 / (a_ref[...] + b_ref[...])


def _rden(d0, d1):
    return pl.pallas_call(
        _rden_body,
        grid=(NPAD // 512,),
        in_specs=[
            pl.BlockSpec((512, 128), lambda i: (i, 0)),
            pl.BlockSpec((512, 128), lambda i: (i, 0)),
        ],
        out_specs=pl.BlockSpec((512, 128), lambda i: (i, 0)),
        out_shape=jax.ShapeDtypeStruct((NPAD, 128), jnp.float32),
    )(d0, d1)


def _pool_body(y_ref, b2_ref, bat_ref, wg1_ref, bg1_ref, wg2_ref, bg2_ref,
               wnp_ref, bnp_ref, gmm_ref, npl_ref, psum, cnt):
    n = pl.program_id(0)

    @pl.when(n == 0)
    def _():
        psum[...] = jnp.zeros_like(psum)
        cnt[...] = jnp.zeros_like(cnt)

    yb = y_ref[...] + b2_ref[...]
    gidx = lax.broadcasted_iota(jnp.int32, (512, 128), 1)
    ohb = (bat_ref[...] == gidx).astype(jnp.float32)
    psum[...] += lax.dot_general(ohb, yb, (((0,), (1,)), ((), ())))
    cnt[...] += lax.dot_general(ohb, jnp.ones((512, 128), jnp.float32),
                                (((0,), (0,)), ((), ())))

    @pl.when(n == NPAD // 512 - 1)
    def _():
        ps = psum[...].reshape(128, HC)
        c1 = cnt[:, 0:1]
        pooled = ps * (1.0 / jnp.maximum(c1, 1.0))
        hh = jnp.maximum(
            jnp.dot(pooled, wg1_ref[...],
                    preferred_element_type=jnp.float32) + bg1_ref[...], 0.0)
        gmm = jnp.dot(hh, wg2_ref[...],
                      preferred_element_type=jnp.float32) + bg2_ref[...]
        npl = jnp.dot(pooled, wnp_ref[...],
                      preferred_element_type=jnp.float32) + bnp_ref[...]
        gmm_ref[...] = gmm[0:8]
        npl_ref[...] = npl[0:8]


def _pool(y2c, b2r, bat2d, wg1, bg1r, wg2p, bg2r, wnpp, bnpr):
    return pl.pallas_call(
        _pool_body,
        grid=(NPAD // 512,),
        in_specs=[
            pl.BlockSpec((8, 512, 128), lambda n: (0, n, 0)),
            pl.BlockSpec((8, 1, 128), lambda n: (0, 0, 0)),
            pl.BlockSpec((512, 128), lambda n: (n, 0)),
            pl.BlockSpec((HC, GH), lambda n: (0, 0)),
            pl.BlockSpec((1, GH), lambda n: (0, 0)),
            pl.BlockSpec((GH, 128), lambda n: (0, 0)),
            pl.BlockSpec((1, 128), lambda n: (0, 0)),
            pl.BlockSpec((HC, 128), lambda n: (0, 0)),
            pl.BlockSpec((1, 128), lambda n: (0, 0)),
        ],
        out_specs=[
            pl.BlockSpec((8, 128), lambda n: (0, 0)),
            pl.BlockSpec((8, 128), lambda n: (0, 0)),
        ],
        out_shape=[
            jax.ShapeDtypeStruct((8, 128), jnp.float32),
            jax.ShapeDtypeStruct((8, 128), jnp.float32),
        ],
        scratch_shapes=[
            pltpu.VMEM((128, 8, 128), jnp.float32),
            pltpu.VMEM((128, 128), jnp.float32),
        ],
    )(y2c, b2r, bat2d, wg1, bg1r, wg2p, bg2r, wnpp, bnpr)


def _pass1_body(xlc, xrc, eec, srch, dsth, atth, ex_out,
                srcv, dstv, bufa, bufb, bufe, exb, alph, attv,
                sema, semb):
    ci = lax.axis_index("c")
    sid = lax.axis_index("s")
    wid = sid * NC + ci
    iota16 = lax.broadcasted_iota(jnp.int32, (16,), 0)
    zeros16 = jnp.zeros((16,), jnp.float32)

    def zrow(r, _):
        for v in range(8):
            exb[r, pl.ds(16 * v, 16)] = zeros16
        return 0

    lax.fori_loop(0, T, zrow, 0)
    pltpu.sync_copy(atth, attv)

    def batch(g, _):
        e0 = wid * PW1 + g * T
        pltpu.sync_copy(srch.at[pl.ds(e0, T)], srcv)
        pltpu.sync_copy(dsth.at[pl.ds(e0, T)], dstv)

        def za(e, _):
            alph[e, :] = zeros16
            return 0

        lax.fori_loop(0, T, za, 0)

        for ch in range(8):
            h = ch // 2
            cpa = pltpu.async_copy(xlc.at[ch].at[srcv], bufa, sema)
            cpb = pltpu.async_copy(xrc.at[ch].at[dstv], bufb, semb)
            pltpu.sync_copy(eec.at[ch].at[pl.ds(e0, T)], bufe)
            cpa.wait()
            cpb.wait()
            attc = [attv[h, pl.ds((ch % 2) * 128 + 16 * v, 16)]
                    for v in range(8)]
            hmask = iota16 == h

            def ebody(e, _):
                acc = zeros16
                for v in range(8):
                    sl = pl.ds(16 * v, 16)
                    t = bufa[e, sl] + bufb[e, sl] + bufe[e, sl]
                    m = jnp.maximum(t, 0.2 * t)
                    acc = acc + m * attc[v]
                s = jnp.sum(acc)
                alph[e, :] = alph[e, :] + jnp.where(hmask, s, 0.0)
                return 0

            lax.fori_loop(0, T, ebody, 0)

        def xbody(e, _):
            exb[e, pl.ds(0, 16)] = jnp.exp(jnp.minimum(alph[e, :], 60.0))
            return 0

        lax.fori_loop(0, T, xbody, 0)
        pltpu.sync_copy(exb, ex_out.at[pl.ds(e0, T)])
        return 0

    lax.fori_loop(0, NB1, batch, 0)


def _pass1(xlc, xrc, eec, src, dst, att):
    f = pl.kernel(
        _pass1_body,
        out_type=jax.ShapeDtypeStruct((E2P, 128), jnp.float32),
        mesh=plsc.VectorSubcoreMesh(
            core_axis_name="c", subcore_axis_name="s",
            num_cores=NC, num_subcores=NS),
        compiler_params=pltpu.CompilerParams(needs_layout_passes=False),
        scratch_types=[
            pltpu.VMEM((T,), jnp.int32),
            pltpu.VMEM((T,), jnp.int32),
            pltpu.VMEM((T, 128), jnp.float32),
            pltpu.VMEM((T, 128), jnp.float32),
            pltpu.VMEM((T, 128), jnp.float32),
            pltpu.VMEM((T, 128), jnp.float32),
            pltpu.VMEM((T, 16), jnp.float32),
            pltpu.VMEM((H, C), jnp.float32),
            pltpu.SemaphoreType.DMA,
            pltpu.SemaphoreType.DMA,
        ],
    )
    return f(xlc, xrc, eec, src, dst, att)


def _den_body(dsth, exh, den_out, dstv, dstl, exb, zb, den_sh):
    ci = lax.axis_index("c")
    sid = lax.axis_index("s")
    wid = sid * NC + ci
    rpt = QNPT // NS

    def zrow(r, _):
        for v in range(8):
            zb[r, pl.ds(16 * v, 16)] = jnp.zeros((16,), jnp.float32)
        return 0

    lax.fori_loop(0, T, zrow, 0)

    for q in range(4):
        base = q * QNP
        r0 = sid * rpt
        pltpu.sync_copy(zb, den_sh.at[pl.ds(r0, T)])
        pltpu.sync_copy(zb.at[0:40], den_sh.at[pl.ds(r0 + T, 40)])
        plsc.subcore_barrier()

        def batch(g, _):
            e0 = wid * PW1 + g * T
            pltpu.sync_copy(dsth.at[pl.ds(e0, T)], dstv)
            pltpu.sync_copy(exh.at[pl.ds(e0, T)], exb)

            def fixd(i, _):
                sl = pl.ds(16 * i, 16)
                d = dstv[sl] - base
                ok = (d >= 0) & (d < QNP)
                dstl[sl] = jnp.where(ok, d, QNP)
                return 0

            lax.fori_loop(0, 8, fixd, 0)
            pltpu.sync_copy(exb, den_sh.at[dstl], add=True)
            return 0

        lax.fori_loop(0, NB1, batch, 0)
        plsc.subcore_barrier()
        c0 = sid * (QNP // NS)
        pltpu.sync_copy(den_sh.at[pl.ds(c0, 128)],
                        den_out.at[ci].at[q].at[pl.ds(c0, 128)])
        pltpu.sync_copy(den_sh.at[pl.ds(c0 + 128, 32)],
                        den_out.at[ci].at[q].at[pl.ds(c0 + 128, 32)])
        plsc.subcore_barrier()


def _den(dst, ex):
    f = pl.kernel(
        _den_body,
        out_type=jax.ShapeDtypeStruct((NC, 4, QNP, 128), jnp.float32),
        mesh=plsc.VectorSubcoreMesh(
            core_axis_name="c", subcore_axis_name="s",
            num_cores=NC, num_subcores=NS),
        compiler_params=pltpu.CompilerParams(needs_layout_passes=False),
        scratch_types=[
            pltpu.VMEM((T,), jnp.int32),
            pltpu.VMEM((T,), jnp.int32),
            pltpu.VMEM((T, 128), jnp.float32),
            pltpu.VMEM((T, 128), jnp.float32),
            pltpu.VMEM_SHARED((QNPT, 128), jnp.float32),
        ],
    )
    return f(dst, ex)


def _pass2_body(xlc, srch, dsth, exh, rdenh, out_hbm,
                srcv, dstv, dstl, buf, exb, rdb, zb, acc_sh, sem):
    ci = lax.axis_index("c")
    sid = lax.axis_index("s")
    iota16 = lax.broadcasted_iota(jnp.int32, (16,), 0)
    zeros16 = jnp.zeros((16,), jnp.float32)

    def zrow(r, _):
        for v in range(8):
            zb[r, pl.ds(16 * v, 16)] = zeros16
        return 0

    lax.fori_loop(0, T, zrow, 0)

    for cc in range(4):
        ch = ci * 4 + cc
        h = ch // 2
        hmask = iota16 == h
        for half in range(2):
            base = half * HNP
            r0 = sid * (HNPT // NS)
            pltpu.sync_copy(zb, acc_sh.at[pl.ds(r0, T)])
            pltpu.sync_copy(zb, acc_sh.at[pl.ds(r0 + T, T)])
            pltpu.sync_copy(zb.at[0:72], acc_sh.at[pl.ds(r0 + 2 * T, 72)])
            plsc.subcore_barrier()

            def batch(g, _):
                e0 = sid * PW2 + g * T
                pltpu.sync_copy(srch.at[pl.ds(e0, T)], srcv)
                pltpu.sync_copy(dsth.at[pl.ds(e0, T)], dstv)
                pltpu.sync_copy(exh.at[pl.ds(e0, T)], exb)
                pltpu.async_copy(rdenh.at[dstv], rdb, sem).wait()
                pltpu.async_copy(xlc.at[ch].at[srcv], buf, sem).wait()

                def fixd(i, _):
                    d = dstv[pl.ds(16 * i, 16)] - base
                    ok = (d >= 0) & (d < HNP)
                    dstl[pl.ds(16 * i, 16)] = jnp.where(ok, d, HNP)
                    return 0

                lax.fori_loop(0, 8, fixd, 0)

                def ebody(e, _):
                    p = exb[e, pl.ds(0, 16)] * rdb[e, pl.ds(0, 16)]
                    a = jnp.sum(jnp.where(hmask, p, 0.0))
                    for v in range(8):
                        sl = pl.ds(16 * v, 16)
                        buf[e, sl] = buf[e, sl] * a
                    return 0

                lax.fori_loop(0, T, ebody, 0)
                pltpu.sync_copy(buf, acc_sh.at[dstl], add=True)
                return 0

            lax.fori_loop(0, NB2, batch, 0)
            plsc.subcore_barrier()
            c0 = sid * (HNP // NS)
            pltpu.sync_copy(acc_sh.at[pl.ds(c0, T)],
                            out_hbm.at[ch].at[pl.ds(base + c0, T)])
            pltpu.sync_copy(acc_sh.at[pl.ds(c0 + T, T)],
                            out_hbm.at[ch].at[pl.ds(base + c0 + T, T)])
            pltpu.sync_copy(acc_sh.at[pl.ds(c0 + 2 * T, 64)],
                            out_hbm.at[ch].at[pl.ds(base + c0 + 2 * T, 64)])
            plsc.subcore_barrier()


def _pass2(xlc, src, dst, ex, rden):
    f = pl.kernel(
        _pass2_body,
        out_type=jax.ShapeDtypeStruct((8, NPAD, 128), jnp.float32),
        mesh=plsc.VectorSubcoreMesh(
            core_axis_name="c", subcore_axis_name="s",
            num_cores=NC, num_subcores=NS),
        compiler_params=pltpu.CompilerParams(needs_layout_passes=False),
        scratch_types=[
            pltpu.VMEM((T,), jnp.int32),
            pltpu.VMEM((T,), jnp.int32),
            pltpu.VMEM((T,), jnp.int32),
            pltpu.VMEM((T, 128), jnp.float32),
            pltpu.VMEM((T, 128), jnp.float32),
            pltpu.VMEM((T, 128), jnp.float32),
            pltpu.VMEM((T, 128), jnp.float32),
            pltpu.VMEM_SHARED((HNPT, 128), jnp.float32),
            pltpu.SemaphoreType.DMA,
        ],
    )
    return f(xlc, src, dst, ex, rden)


def _gat_layer(x_in, eec, src, dst, w_lr, att):
    xall = _proj(x_in, w_lr, 16, 1024)
    xlc, xrc = xall[:8], xall[8:]
    ex = _pass1(xlc, xrc, eec, src, dst, att)
    den = _den(dst, ex)
    d0 = den[0].reshape(NPAD, 128)
    d1 = den[1].reshape(NPAD, 128)
    rden = _rden(d0, d1)
    outc = _pass2(xlc, src, dst, ex, rden)
    return outc


def kernel(x, edge_index, edge_attr, batch, Wl1, Wr1, We1, att1, b1,
           Wl2, Wr2, We2, att2, b2, Wg1, bg1, Wg2, bg2, Wnp, bnp):
    f32 = jnp.float32
    x = x.astype(f32)

    src = jnp.concatenate([edge_index[0].astype(jnp.int32),
                           jnp.arange(N, dtype=jnp.int32),
                           jnp.full((E2P - E2,), DUMMY, jnp.int32)])
    dst = jnp.concatenate([edge_index[1].astype(jnp.int32),
                           jnp.arange(N, dtype=jnp.int32),
                           jnp.full((E2P - E2,), DUMMY, jnp.int32)])

    ea128 = jnp.zeros((EP1, 128), f32).at[:E, :ED].set(edge_attr)
    emean = _colmean(ea128)[0:1, :]

    ea_full = jnp.concatenate([
        ea128[:E],
        jnp.broadcast_to(emean, (N, 128)),
        jnp.zeros((E2P - E2, 128), f32),
    ])

    xpad = jnp.zeros((NPAD, D), f32).at[:N].set(x)

    w12 = jnp.concatenate([Wl1, Wr1], axis=1)
    we1p = jnp.zeros((128, HC), f32).at[:ED].set(We1)
    we2p = jnp.zeros((128, HC), f32).at[:ED].set(We2)

    eec1 = _proj(ea_full, we1p, 8, 2048)
    out1 = _gat_layer(xpad, eec1, src, dst, w12, att1)

    b1r = b1.reshape(8, 1, 128)
    yc = _elu(out1, b1r)

    w34 = jnp.concatenate([Wl2, Wr2], axis=1)
    eec2 = _proj(ea_full, we2p, 8, 2048)
    out2 = _gat_layer(yc, eec2, src, dst, w34, att2)

    b2r = b2.reshape(8, 1, 128)
    batp = jnp.concatenate(
        [batch.astype(jnp.int32), jnp.full((NPAD - N,), B, jnp.int32)])
    bat2d = jnp.broadcast_to(batp[:, None], (NPAD, 128))
    bg1r = bg1.reshape(1, GH)
    wg2p = jnp.zeros((GH, 128), f32).at[:, :NM * 5].set(Wg2)
    bg2r = jnp.zeros((1, 128), f32).at[0, :NM * 5].set(bg2)
    wnpp = jnp.zeros((HC, 128), f32).at[:, :AD].set(Wnp)
    bnpr = jnp.zeros((1, 128), f32).at[0, :AD].set(bnp)

    gmm_p, npl_p = _pool(out2, b2r, bat2d, Wg1, bg1r, wg2p, bg2r, wnpp, bnpr)
    return (gmm_p[:, :NM * 5], npl_p[:, :AD])
